# baseline (speedup 1.0000x reference)
import functools
import jax
import jax.numpy as jnp
from jax import lax
from jax.experimental import pallas as pl
from jax.experimental.pallas import tpu as pltpu, tpu_sc as plsc


def _add_kernel(x_ref, pe_ref, o_ref):
    o_ref[...] = x_ref[...] + pe_ref[...]


def _tc_part(x, pe, n_batch, bs):
    B, S, D = x.shape
    grid = (S // bs, n_batch)
    return pl.pallas_call(
        _add_kernel,
        grid=grid,
        in_specs=[
            pl.BlockSpec((1, bs, D), lambda i, j: (j, i, 0)),
            pl.BlockSpec((bs, D), lambda i, j: (i, 0)),
        ],
        out_specs=pl.BlockSpec((1, bs, D), lambda i, j: (j, i, 0)),
        out_shape=jax.ShapeDtypeStruct((n_batch, S, D), x.dtype),
    )(x, pe)


def _sc_part(x_flat, pe_flat, elem_base, n_elems):
    NC, NS = 2, 16
    NW = NC * NS
    PER_W = n_elems // NW
    CH = 32768
    N_CHUNKS = PER_W // CH

    mesh = plsc.VectorSubcoreMesh(core_axis_name="c", subcore_axis_name="s")

    @functools.partial(
        pl.kernel,
        mesh=mesh,
        out_type=jax.ShapeDtypeStruct((n_elems,), jnp.float32),
        scratch_types=[
            pltpu.VMEM((CH,), jnp.float32),
            pltpu.VMEM((CH,), jnp.float32),
        ],
    )
    def k(x_hbm, pe_hbm, o_hbm, xbuf, pebuf):
        wid = lax.axis_index("s") * NC + lax.axis_index("c")
        base = wid * PER_W

        def chunk_body(ci, _):
            off = base + ci * CH
            pltpu.sync_copy(x_hbm.at[pl.ds(elem_base + off, CH)], xbuf)
            pltpu.sync_copy(pe_hbm.at[pl.ds(off, CH)], pebuf)

            @plsc.parallel_loop(0, CH // 16, unroll=8)
            def vec_body(i):
                sl = pl.ds(i * 16, 16)
                xbuf[sl] = xbuf[sl] + pebuf[sl]

            pltpu.sync_copy(xbuf, o_hbm.at[pl.ds(off, CH)])
            return 0

        lax.fori_loop(0, N_CHUNKS, chunk_body, 0)

    return k(x_flat, pe_flat)


def kernel(x, pe):
    B, S, D = x.shape
    B_TC = B - 1
    tc_out = _tc_part(x, pe[:S], B_TC, 2048)
    sc_out = _sc_part(x.reshape(B * S * D), pe[:S].reshape(S * D),
                      B_TC * S * D, S * D)
    flat = jnp.concatenate([tc_out.reshape(B_TC * S * D), sc_out], axis=0)
    return flat.reshape(B, S, D)


# speedup vs baseline: 6.0535x; 6.0535x over previous
import jax
import jax.numpy as jnp
from jax.experimental import pallas as pl
from jax.experimental.pallas import tpu as pltpu


def _add_kernel(x_ref, pe_ref, o_ref):
    o_ref[...] = x_ref[...] + pe_ref[...]


def kernel(x, pe):
    B, S, D = x.shape
    BS = 2048
    grid = (S // BS, B)
    return pl.pallas_call(
        _add_kernel,
        grid=grid,
        in_specs=[
            pl.BlockSpec((1, BS, D), lambda i, j: (j, i, 0)),
            pl.BlockSpec((BS, D), lambda i, j: (i, 0)),
        ],
        out_specs=pl.BlockSpec((1, BS, D), lambda i, j: (j, i, 0)),
        out_shape=jax.ShapeDtypeStruct((B, S, D), x.dtype),
        compiler_params=pltpu.CompilerParams(vmem_limit_bytes=128 * 1024 * 1024),
    )(x, pe[:S])
